# baseline (speedup 1.0000x reference)
import dataclasses
import functools

import jax
import jax.numpy as jnp
from jax import lax
from jax.experimental import pallas as pl
from jax.experimental.pallas import tpu as pltpu
from jax.experimental.pallas import tpu_sc as plsc

_NC, _NS = 2, 16
_NW = _NC * _NS
_L = 16


def _sc_mesh():
    return plsc.VectorSubcoreMesh(core_axis_name="c", subcore_axis_name="s")


def _sc_params():
    cp = pltpu.CompilerParams()
    fields = pltpu.CompilerParams.__dataclass_fields__
    if "needs_layout_passes" in fields:
        cp = dataclasses.replace(cp, needs_layout_passes=False)
    if "use_tc_tiling_on_sc" in fields:
        cp = dataclasses.replace(cp, use_tc_tiling_on_sc=False)
    return cp


def _wid():
    return lax.axis_index("s") * _NC + lax.axis_index("c")


def _deg_body(src_hbm, ew_hbm, deg_hbm, sbuf, wbuf, degbuf, *, E, NP, EB):
    @pl.when(_wid() == 0)
    def _():
        zeros = jnp.zeros((_L,), jnp.float32)

        @pl.loop(0, NP, step=_L)
        def _(i):
            degbuf[pl.ds(i, _L)] = zeros

        @pl.loop(0, E, step=EB)
        def _(e0):
            pltpu.sync_copy(src_hbm.at[pl.ds(e0, EB)], sbuf)
            pltpu.sync_copy(ew_hbm.at[pl.ds(e0, EB)], wbuf)

            @plsc.parallel_loop(0, EB, step=_L, unroll=2)
            def _(j):
                s16 = sbuf[pl.ds(j, _L)]
                w16 = wbuf[pl.ds(j, _L)]
                plsc.addupdate_scatter(degbuf, [s16], w16)

        pltpu.sync_copy(degbuf, deg_hbm)


def _run_deg(src_p, ew_p, NP, EB=8192):
    E = src_p.shape[0]
    k = pl.kernel(
        functools.partial(_deg_body, E=E, NP=NP, EB=EB),
        out_type=jax.ShapeDtypeStruct((NP,), jnp.float32),
        mesh=_sc_mesh(),
        compiler_params=_sc_params(),
        scratch_types=[
            pltpu.VMEM((EB,), jnp.int32),
            pltpu.VMEM((EB,), jnp.float32),
            pltpu.VMEM((NP,), jnp.float32),
        ],
    )
    return k(src_p, ew_p)


def _dis_body(deg_ref, o_ref):
    d = deg_ref[...]
    o_ref[...] = jnp.where(d > 0, lax.rsqrt(jnp.where(d > 0, d, 1.0)), 0.0)


def _run_dis(deg, NP):
    d2 = deg.reshape(NP // 128, 128)
    out = pl.pallas_call(
        _dis_body,
        out_shape=jax.ShapeDtypeStruct((NP // 128, 128), jnp.float32),
    )(d2)
    return out.reshape(NP)


def _wnorm_body(dis_hbm, src_hbm, dst_hbm, ew_hbm, wn_hbm,
                disbuf, sbuf, dbuf, wbuf, obuf, *, EW):
    base = _wid() * EW
    pltpu.sync_copy(dis_hbm, disbuf)
    pltpu.sync_copy(src_hbm.at[pl.ds(base, EW)], sbuf)
    pltpu.sync_copy(dst_hbm.at[pl.ds(base, EW)], dbuf)
    pltpu.sync_copy(ew_hbm.at[pl.ds(base, EW)], wbuf)

    @plsc.parallel_loop(0, EW, step=_L, unroll=2)
    def _(j):
        s16 = sbuf[pl.ds(j, _L)]
        d16 = dbuf[pl.ds(j, _L)]
        w16 = wbuf[pl.ds(j, _L)]
        a = plsc.load_gather(disbuf, [s16])
        b = plsc.load_gather(disbuf, [d16])
        obuf[pl.ds(j, _L)] = -(a * w16) * b

    pltpu.sync_copy(obuf, wn_hbm.at[pl.ds(base, EW)])


def _run_wnorm(dis, src_p, dst_p, ew_p, NP):
    E = src_p.shape[0]
    EW = E // _NW
    k = pl.kernel(
        functools.partial(_wnorm_body, EW=EW),
        out_type=jax.ShapeDtypeStruct((E,), jnp.float32),
        mesh=_sc_mesh(),
        compiler_params=_sc_params(),
        scratch_types=[
            pltpu.VMEM((NP,), jnp.float32),
            pltpu.VMEM((EW,), jnp.int32),
            pltpu.VMEM((EW,), jnp.int32),
            pltpu.VMEM((EW,), jnp.float32),
            pltpu.VMEM((EW,), jnp.float32),
        ],
    )
    return k(dis, src_p, dst_p, ew_p)


def _dg(a, b):
    return lax.dot_general(a, b, (((0,), (0,)), ((), ())),
                           preferred_element_type=jnp.float32)


def _tconv1_body(x0_ref, x1_ref, x2_ref, w_ref, b_ref, o_ref, *, H):
    w = w_ref[...].astype(jnp.bfloat16)
    acc = None
    for k, xr in enumerate((x0_ref, x1_ref, x2_ref)):
        xk = xr[0].astype(jnp.bfloat16)
        p = _dg(w[k], xk)
        acc = p if acc is None else acc + p
    acc = acc + b_ref[...]
    a, g, c = acc[:H], acc[H:2 * H], acc[2 * H:]
    o_ref[...] = jax.nn.relu(a + jax.nn.sigmoid(g) + c)


def _run_tconv1(Xp, Wc, bc, T1, H, NP, bw=512):
    KS, C, _ = Wc.shape
    grid = (T1, NP // bw)
    in_specs = [
        pl.BlockSpec((1, C, bw), (lambda t, n, k=k: (t + k, 0, n)))
        for k in range(KS)
    ] + [
        pl.BlockSpec((KS, C, 3 * H), lambda t, n: (0, 0, 0)),
        pl.BlockSpec((3 * H, 1), lambda t, n: (0, 0)),
    ]
    out = pl.pallas_call(
        functools.partial(_tconv1_body, H=H),
        grid=grid,
        in_specs=in_specs,
        out_specs=pl.BlockSpec((H, bw), lambda t, n: (t, n)),
        out_shape=jax.ShapeDtypeStruct((T1 * H, NP), jnp.float32),
    )(Xp, Xp, Xp, Wc, bc)
    return out


def _cheb_mp_body(x_hbm, ep_hbm, y1_hbm, y2_hbm, xbuf, ybuf, ebuf,
                  *, NP, EB, NBLK, R, G):
    wid = _wid()

    def mp_pass(src_ref, acc_ref):
        zeros = jnp.zeros((_L,), jnp.float32)
        for r in range(R):
            @pl.loop(0, NP, step=_L)
            def _(i):
                acc_ref[r, pl.ds(i, _L)] = zeros

        @pl.loop(0, NBLK)
        def _(blk):
            pltpu.sync_copy(ep_hbm.at[blk], ebuf)

            @plsc.parallel_loop(0, EB, step=_L, unroll=4)
            def _(j):
                s16 = ebuf[0, pl.ds(j, _L)]
                d16 = ebuf[1, pl.ds(j, _L)]
                w16 = plsc.bitcast(ebuf[2, pl.ds(j, _L)], jnp.float32)
                for r in range(R):
                    r16 = jnp.full((_L,), r, jnp.int32)
                    zv = plsc.load_gather(src_ref, [r16, s16])
                    plsc.addupdate_scatter(acc_ref, [r16, d16], w16 * zv)

    for g in range(G):
        row0 = wid * (R * G) + g * R
        pltpu.sync_copy(x_hbm.at[pl.ds(row0, R)], xbuf)
        mp_pass(xbuf, ybuf)
        pltpu.sync_copy(ybuf, y1_hbm.at[pl.ds(row0, R)])
        mp_pass(ybuf, xbuf)
        pltpu.sync_copy(xbuf, y2_hbm.at[pl.ds(row0, R)])


def _run_cheb_mp(Tx0, ep, NP, R, G, EB):
    NBLK = ep.shape[0]
    RT = Tx0.shape[0]
    sds = jax.ShapeDtypeStruct((RT, NP), jnp.float32)
    k = pl.kernel(
        functools.partial(_cheb_mp_body, NP=NP, EB=EB, NBLK=NBLK, R=R, G=G),
        out_type=(sds, sds),
        mesh=_sc_mesh(),
        compiler_params=_sc_params(),
        scratch_types=[
            pltpu.VMEM((R, NP), jnp.float32),
            pltpu.VMEM((R, NP), jnp.float32),
            pltpu.VMEM((3, EB), jnp.int32),
        ],
    )
    return k(Tx0, ep)


def _tail_body(tx0_ref, y1_ref, y2_ref, chw_ref, chb_ref, w2_ref, b2_ref,
               g_ref, bt_ref, o_ref, *, T1, T2, H, F):
    A0 = (chw_ref[0] - chw_ref[2]).astype(jnp.bfloat16)
    A1 = chw_ref[1].astype(jnp.bfloat16)
    A2 = (2.0 * chw_ref[2]).astype(jnp.bfloat16)
    chb = chb_ref[...]

    t2 = []
    for t in range(T1):
        z0 = tx0_ref[t * H:(t + 1) * H, :].astype(jnp.bfloat16)
        z1 = y1_ref[t * H:(t + 1) * H, :].astype(jnp.bfloat16)
        z2 = y2_ref[t * H:(t + 1) * H, :].astype(jnp.bfloat16)
        t2.append(jax.nn.relu(_dg(A0, z0) + _dg(A1, z1) + _dg(A2, z2) + chb))

    w2 = w2_ref[...].astype(jnp.bfloat16)
    b2 = b2_ref[...]
    t3 = []
    for tt in range(T2):
        acc = None
        for k in range(w2.shape[0]):
            p = _dg(w2[k], t2[tt + k].astype(jnp.bfloat16))
            acc = p if acc is None else acc + p
        acc = acc + b2
        a, g, c = acc[:F], acc[F:2 * F], acc[2 * F:]
        t3.append(jax.nn.relu(a + jax.nn.sigmoid(g) + c))

    cnt = float(T2 * F)
    m = sum(jnp.sum(x, axis=0) for x in t3) / cnt
    v = sum(jnp.sum((x - m[None, :]) ** 2, axis=0) for x in t3) / cnt
    scale = lax.rsqrt(v + 1e-5) * g_ref[0]
    bet = bt_ref[0]
    for tt in range(T2):
        o_ref[tt * F:(tt + 1) * F, :] = (t3[tt] - m[None, :]) * scale[None, :] + bet[None, :]


def _run_tail(Tx0, Y1, Y2, cheb_w, chb, Wc2, bc2, bn_g, bn_b,
              T1, T2, H, F, NP, bw=512):
    RT = Tx0.shape[0]
    KS = Wc2.shape[0]
    fm = pl.BlockSpec((RT, bw), lambda n: (0, n))
    full = lambda shape: pl.BlockSpec(shape, lambda n: tuple(0 for _ in shape))
    out = pl.pallas_call(
        functools.partial(_tail_body, T1=T1, T2=T2, H=H, F=F),
        grid=(NP // bw,),
        in_specs=[
            fm, fm, fm,
            full((cheb_w.shape[0], H, H)),
            full((H, 1)),
            full((KS, H, 3 * F)),
            full((3 * F, 1)),
            pl.BlockSpec((1, bw), lambda n: (0, n)),
            pl.BlockSpec((1, bw), lambda n: (0, n)),
        ],
        out_specs=pl.BlockSpec((T2 * F, bw), lambda n: (0, n)),
        out_shape=jax.ShapeDtypeStruct((T2 * F, NP), jnp.float32),
    )(Tx0, Y1, Y2, cheb_w, chb, Wc2, bc2, bn_g, bn_b)
    return out


def _round_up(x, m):
    return (x + m - 1) // m * m


def _conv_wstack(w1, w2, w3):
    return jnp.concatenate(
        [jnp.transpose(w[:, :, 0, :], (2, 1, 0)) for w in (w1, w2, w3)],
        axis=2)


def kernel(X, edge_index, edge_weight, tc1_w1, tc1_b1, tc1_w2, tc1_b2,
           tc1_w3, tc1_b3, cheb_w, cheb_b, tc2_w1, tc2_b1, tc2_w2, tc2_b2,
           tc2_w3, tc2_b3, bn_gamma, bn_beta):
    B, T, N, C = X.shape
    H = tc1_w1.shape[0]
    F = tc2_w1.shape[0]
    T1, T2 = T - 2, T - 4
    E = edge_weight.shape[0]

    NP = _round_up(N, 512)
    EB_DEG, EB_MP = 8192, 4096
    EP = _round_up(E, max(EB_DEG, EB_MP, _NW * _L * 8))

    src_p = jnp.pad(edge_index[0], (0, EP - E))
    dst_p = jnp.pad(edge_index[1], (0, EP - E))
    ew_p = jnp.pad(edge_weight, (0, EP - E))

    deg = _run_deg(src_p, ew_p, NP, EB=EB_DEG)
    dis = _run_dis(deg, NP)
    wn = _run_wnorm(dis, src_p, dst_p, ew_p, NP)

    wn_i = lax.bitcast_convert_type(wn, jnp.int32)
    ep = jnp.stack([src_p, dst_p, wn_i]).reshape(3, EP // EB_MP, EB_MP)
    ep = jnp.transpose(ep, (1, 0, 2))

    Xp = jnp.pad(jnp.transpose(X[0], (0, 2, 1)), ((0, 0), (0, 0), (0, NP - N)))
    Wc1 = _conv_wstack(tc1_w1, tc1_w2, tc1_w3)
    bc1 = jnp.concatenate([tc1_b1, tc1_b2, tc1_b3])[:, None]
    Tx0 = _run_tconv1(Xp, Wc1, bc1, T1, H, NP)

    RT = T1 * H
    rows_per_w = RT // _NW
    R = 5 if rows_per_w % 5 == 0 else rows_per_w
    G = rows_per_w // R
    Y1, Y2 = _run_cheb_mp(Tx0, ep, NP, R, G, EB_MP)

    Wc2 = _conv_wstack(tc2_w1, tc2_w2, tc2_w3)
    bc2 = jnp.concatenate([tc2_b1, tc2_b2, tc2_b3])[:, None]
    bn_g = jnp.pad(bn_gamma, (0, NP - N))[None, :]
    bn_b = jnp.pad(bn_beta, (0, NP - N))[None, :]
    out_fm = _run_tail(Tx0, Y1, Y2, cheb_w, cheb_b[:, None], Wc2, bc2,
                       bn_g, bn_b, T1, T2, H, F, NP)

    out = out_fm[:, :N].reshape(T2, F, N)
    return jnp.transpose(out, (0, 2, 1))[None]


# speedup vs baseline: 1.2036x; 1.2036x over previous
import dataclasses
import functools

import jax
import jax.numpy as jnp
from jax import lax
from jax.experimental import pallas as pl
from jax.experimental.pallas import tpu as pltpu
from jax.experimental.pallas import tpu_sc as plsc

_NC, _NS = 2, 16
_NW = _NC * _NS
_L = 16


def _sc_mesh():
    return plsc.VectorSubcoreMesh(core_axis_name="c", subcore_axis_name="s")


def _sc_params():
    cp = pltpu.CompilerParams()
    fields = pltpu.CompilerParams.__dataclass_fields__
    if "needs_layout_passes" in fields:
        cp = dataclasses.replace(cp, needs_layout_passes=False)
    if "use_tc_tiling_on_sc" in fields:
        cp = dataclasses.replace(cp, use_tc_tiling_on_sc=False)
    return cp


def _wid():
    return lax.axis_index("s") * _NC + lax.axis_index("c")


def _deg_body(src_hbm, ew_hbm, deg_hbm, sbuf, wbuf, degbuf, *, E, NP, EB):
    @pl.when(_wid() == 0)
    def _():
        zeros = jnp.zeros((_L,), jnp.float32)

        @pl.loop(0, NP, step=_L)
        def _(i):
            degbuf[pl.ds(i, _L)] = zeros

        @pl.loop(0, E, step=EB)
        def _(e0):
            pltpu.sync_copy(src_hbm.at[pl.ds(e0, EB)], sbuf)
            pltpu.sync_copy(ew_hbm.at[pl.ds(e0, EB)], wbuf)

            @plsc.parallel_loop(0, EB, step=_L, unroll=2)
            def _(j):
                s16 = sbuf[pl.ds(j, _L)]
                w16 = wbuf[pl.ds(j, _L)]
                plsc.addupdate_scatter(degbuf, [s16], w16)

        pltpu.sync_copy(degbuf, deg_hbm)


def _run_deg(src_p, ew_p, NP, EB=8192):
    E = src_p.shape[0]
    k = pl.kernel(
        functools.partial(_deg_body, E=E, NP=NP, EB=EB),
        out_type=jax.ShapeDtypeStruct((NP,), jnp.float32),
        mesh=_sc_mesh(),
        compiler_params=_sc_params(),
        scratch_types=[
            pltpu.VMEM((EB,), jnp.int32),
            pltpu.VMEM((EB,), jnp.float32),
            pltpu.VMEM((NP,), jnp.float32),
        ],
    )
    return k(src_p, ew_p)


def _dis_body(deg_ref, o_ref):
    d = deg_ref[...]
    o_ref[...] = jnp.where(d > 0, lax.rsqrt(jnp.where(d > 0, d, 1.0)), 0.0)


def _run_dis(deg, NP):
    d2 = deg.reshape(NP // 128, 128)
    out = pl.pallas_call(
        _dis_body,
        out_shape=jax.ShapeDtypeStruct((NP // 128, 128), jnp.float32),
    )(d2)
    return out.reshape(NP)


def _wnorm_body(dis_hbm, src_hbm, dst_hbm, ew_hbm, wn_hbm,
                disbuf, sbuf, dbuf, wbuf, obuf, *, EW):
    base = _wid() * EW
    pltpu.sync_copy(dis_hbm, disbuf)
    pltpu.sync_copy(src_hbm.at[pl.ds(base, EW)], sbuf)
    pltpu.sync_copy(dst_hbm.at[pl.ds(base, EW)], dbuf)
    pltpu.sync_copy(ew_hbm.at[pl.ds(base, EW)], wbuf)

    @plsc.parallel_loop(0, EW, step=_L, unroll=2)
    def _(j):
        s16 = sbuf[pl.ds(j, _L)]
        d16 = dbuf[pl.ds(j, _L)]
        w16 = wbuf[pl.ds(j, _L)]
        a = plsc.load_gather(disbuf, [s16])
        b = plsc.load_gather(disbuf, [d16])
        obuf[pl.ds(j, _L)] = -(a * w16) * b

    pltpu.sync_copy(obuf, wn_hbm.at[pl.ds(base, EW)])


def _run_wnorm(dis, src_p, dst_p, ew_p, NP):
    E = src_p.shape[0]
    EW = E // _NW
    k = pl.kernel(
        functools.partial(_wnorm_body, EW=EW),
        out_type=jax.ShapeDtypeStruct((E,), jnp.float32),
        mesh=_sc_mesh(),
        compiler_params=_sc_params(),
        scratch_types=[
            pltpu.VMEM((NP,), jnp.float32),
            pltpu.VMEM((EW,), jnp.int32),
            pltpu.VMEM((EW,), jnp.int32),
            pltpu.VMEM((EW,), jnp.float32),
            pltpu.VMEM((EW,), jnp.float32),
        ],
    )
    return k(dis, src_p, dst_p, ew_p)


def _dg(a, b):
    return lax.dot_general(a, b, (((0,), (0,)), ((), ())),
                           preferred_element_type=jnp.float32)


def _tconv1_body(x0_ref, x1_ref, x2_ref, w_ref, b_ref, o_ref, *, H):
    w = w_ref[...].astype(jnp.bfloat16)
    acc = None
    for k, xr in enumerate((x0_ref, x1_ref, x2_ref)):
        xk = xr[0].astype(jnp.bfloat16)
        p = _dg(w[k], xk)
        acc = p if acc is None else acc + p
    acc = acc + b_ref[...]
    a, g, c = acc[:H], acc[H:2 * H], acc[2 * H:]
    o_ref[...] = jax.nn.relu(a + jax.nn.sigmoid(g) + c)


def _run_tconv1(Xp, Wc, bc, T1, H, NP, bw=512):
    KS, C, _ = Wc.shape
    grid = (T1, NP // bw)
    in_specs = [
        pl.BlockSpec((1, C, bw), (lambda t, n, k=k: (t + k, 0, n)))
        for k in range(KS)
    ] + [
        pl.BlockSpec((KS, C, 3 * H), lambda t, n: (0, 0, 0)),
        pl.BlockSpec((3 * H, 1), lambda t, n: (0, 0)),
    ]
    out = pl.pallas_call(
        functools.partial(_tconv1_body, H=H),
        grid=grid,
        in_specs=in_specs,
        out_specs=pl.BlockSpec((H, bw), lambda t, n: (t, n)),
        out_shape=jax.ShapeDtypeStruct((T1 * H, NP), jnp.float32),
    )(Xp, Xp, Xp, Wc, bc)
    return out


def _cheb_mp_body(x_hbm, ep_hbm, y1_hbm, y2_hbm, xbuf, ybuf, eb0, eb1,
                  sem0, sem1, *, NP, EB, NBLK, R, G):
    wid = _wid()

    def mp_pass(src_ref, acc_ref):
        zeros = jnp.zeros((_L,), jnp.float32)
        for r in range(R):
            @pl.loop(0, NP, step=_L)
            def _(i):
                acc_ref[r, pl.ds(i, _L)] = zeros

        def process(ebuf):
            @plsc.parallel_loop(0, EB, step=_L, unroll=2)
            def _(j):
                s16 = ebuf[0, pl.ds(j, _L)]
                d16 = ebuf[1, pl.ds(j, _L)]
                w16 = plsc.bitcast(ebuf[2, pl.ds(j, _L)], jnp.float32)
                for r in range(R):
                    r16 = jnp.full((_L,), r, jnp.int32)
                    zv = plsc.load_gather(src_ref, [r16, s16])
                    plsc.addupdate_scatter(acc_ref, [r16, d16], w16 * zv)

        pltpu.async_copy(ep_hbm.at[0], eb0, sem0)

        @pl.loop(0, NBLK, step=2)
        def _(blk):
            pltpu.async_copy(ep_hbm.at[blk + 1], eb1, sem1)
            pltpu.make_async_copy(ep_hbm.at[blk], eb0, sem0).wait()
            process(eb0)

            @pl.when(blk + 2 < NBLK)
            def _():
                pltpu.async_copy(ep_hbm.at[blk + 2], eb0, sem0)

            pltpu.make_async_copy(ep_hbm.at[blk + 1], eb1, sem1).wait()
            process(eb1)

    for g in range(G):
        row0 = wid * (R * G) + g * R
        pltpu.sync_copy(x_hbm.at[pl.ds(row0, R)], xbuf)
        mp_pass(xbuf, ybuf)
        pltpu.sync_copy(ybuf, y1_hbm.at[pl.ds(row0, R)])
        mp_pass(ybuf, xbuf)
        pltpu.sync_copy(xbuf, y2_hbm.at[pl.ds(row0, R)])


def _run_cheb_mp(Tx0, ep, NP, R, G, EB):
    NBLK = ep.shape[0]
    RT = Tx0.shape[0]
    sds = jax.ShapeDtypeStruct((RT, NP), jnp.float32)
    k = pl.kernel(
        functools.partial(_cheb_mp_body, NP=NP, EB=EB, NBLK=NBLK, R=R, G=G),
        out_type=(sds, sds),
        mesh=_sc_mesh(),
        compiler_params=_sc_params(),
        scratch_types=[
            pltpu.VMEM((R, NP), jnp.float32),
            pltpu.VMEM((R, NP), jnp.float32),
            pltpu.VMEM((3, EB), jnp.int32),
            pltpu.VMEM((3, EB), jnp.int32),
            pltpu.SemaphoreType.DMA,
            pltpu.SemaphoreType.DMA,
        ],
    )
    return k(Tx0, ep)


def _tail_body(tx0_ref, y1_ref, y2_ref, chw_ref, chb_ref, w2_ref, b2_ref,
               g_ref, bt_ref, o_ref, *, T1, T2, H, F):
    A0 = (chw_ref[0] - chw_ref[2]).astype(jnp.bfloat16)
    A1 = chw_ref[1].astype(jnp.bfloat16)
    A2 = (2.0 * chw_ref[2]).astype(jnp.bfloat16)
    chb = chb_ref[...]

    t2 = []
    for t in range(T1):
        z0 = tx0_ref[t * H:(t + 1) * H, :].astype(jnp.bfloat16)
        z1 = y1_ref[t * H:(t + 1) * H, :].astype(jnp.bfloat16)
        z2 = y2_ref[t * H:(t + 1) * H, :].astype(jnp.bfloat16)
        t2.append(jax.nn.relu(_dg(A0, z0) + _dg(A1, z1) + _dg(A2, z2) + chb))

    w2 = w2_ref[...].astype(jnp.bfloat16)
    b2 = b2_ref[...]
    t3 = []
    for tt in range(T2):
        acc = None
        for k in range(w2.shape[0]):
            p = _dg(w2[k], t2[tt + k].astype(jnp.bfloat16))
            acc = p if acc is None else acc + p
        acc = acc + b2
        a, g, c = acc[:F], acc[F:2 * F], acc[2 * F:]
        t3.append(jax.nn.relu(a + jax.nn.sigmoid(g) + c))

    cnt = float(T2 * F)
    m = sum(jnp.sum(x, axis=0) for x in t3) / cnt
    v = sum(jnp.sum((x - m[None, :]) ** 2, axis=0) for x in t3) / cnt
    scale = lax.rsqrt(v + 1e-5) * g_ref[0]
    bet = bt_ref[0]
    for tt in range(T2):
        o_ref[tt * F:(tt + 1) * F, :] = (t3[tt] - m[None, :]) * scale[None, :] + bet[None, :]


def _run_tail(Tx0, Y1, Y2, cheb_w, chb, Wc2, bc2, bn_g, bn_b,
              T1, T2, H, F, NP, bw=512):
    RT = Tx0.shape[0]
    KS = Wc2.shape[0]
    fm = pl.BlockSpec((RT, bw), lambda n: (0, n))
    full = lambda shape: pl.BlockSpec(shape, lambda n: tuple(0 for _ in shape))
    out = pl.pallas_call(
        functools.partial(_tail_body, T1=T1, T2=T2, H=H, F=F),
        grid=(NP // bw,),
        in_specs=[
            fm, fm, fm,
            full((cheb_w.shape[0], H, H)),
            full((H, 1)),
            full((KS, H, 3 * F)),
            full((3 * F, 1)),
            pl.BlockSpec((1, bw), lambda n: (0, n)),
            pl.BlockSpec((1, bw), lambda n: (0, n)),
        ],
        out_specs=pl.BlockSpec((T2 * F, bw), lambda n: (0, n)),
        out_shape=jax.ShapeDtypeStruct((T2 * F, NP), jnp.float32),
    )(Tx0, Y1, Y2, cheb_w, chb, Wc2, bc2, bn_g, bn_b)
    return out


def _round_up(x, m):
    return (x + m - 1) // m * m


def _conv_wstack(w1, w2, w3):
    return jnp.concatenate(
        [jnp.transpose(w[:, :, 0, :], (2, 1, 0)) for w in (w1, w2, w3)],
        axis=2)


def kernel(X, edge_index, edge_weight, tc1_w1, tc1_b1, tc1_w2, tc1_b2,
           tc1_w3, tc1_b3, cheb_w, cheb_b, tc2_w1, tc2_b1, tc2_w2, tc2_b2,
           tc2_w3, tc2_b3, bn_gamma, bn_beta):
    B, T, N, C = X.shape
    H = tc1_w1.shape[0]
    F = tc2_w1.shape[0]
    T1, T2 = T - 2, T - 4
    E = edge_weight.shape[0]

    NP = _round_up(N, 512)
    EB_DEG, EB_MP = 8192, 2048
    EP = _round_up(E, max(EB_DEG, EB_MP, _NW * _L * 8))

    src_p = jnp.pad(edge_index[0], (0, EP - E))
    dst_p = jnp.pad(edge_index[1], (0, EP - E))
    ew_p = jnp.pad(edge_weight, (0, EP - E))

    deg = _run_deg(src_p, ew_p, NP, EB=EB_DEG)
    dis = _run_dis(deg, NP)
    wn = _run_wnorm(dis, src_p, dst_p, ew_p, NP)

    wn_i = lax.bitcast_convert_type(wn, jnp.int32)
    ep = jnp.stack([src_p, dst_p, wn_i]).reshape(3, EP // EB_MP, EB_MP)
    ep = jnp.transpose(ep, (1, 0, 2))

    Xp = jnp.pad(jnp.transpose(X[0], (0, 2, 1)), ((0, 0), (0, 0), (0, NP - N)))
    Wc1 = _conv_wstack(tc1_w1, tc1_w2, tc1_w3)
    bc1 = jnp.concatenate([tc1_b1, tc1_b2, tc1_b3])[:, None]
    Tx0 = _run_tconv1(Xp, Wc1, bc1, T1, H, NP)

    RT = T1 * H
    rows_per_w = RT // _NW
    R = 5 if rows_per_w % 5 == 0 else rows_per_w
    G = rows_per_w // R
    Y1, Y2 = _run_cheb_mp(Tx0, ep, NP, R, G, EB_MP)

    Wc2 = _conv_wstack(tc2_w1, tc2_w2, tc2_w3)
    bc2 = jnp.concatenate([tc2_b1, tc2_b2, tc2_b3])[:, None]
    bn_g = jnp.pad(bn_gamma, (0, NP - N))[None, :]
    bn_b = jnp.pad(bn_beta, (0, NP - N))[None, :]
    out_fm = _run_tail(Tx0, Y1, Y2, cheb_w, cheb_b[:, None], Wc2, bc2,
                       bn_g, bn_b, T1, T2, H, F, NP)

    out = out_fm[:, :N].reshape(T2, F, N)
    return jnp.transpose(out, (0, 2, 1))[None]


# speedup vs baseline: 1.2880x; 1.0702x over previous
import dataclasses
import functools

import jax
import jax.numpy as jnp
from jax import lax
from jax.experimental import pallas as pl
from jax.experimental.pallas import tpu as pltpu
from jax.experimental.pallas import tpu_sc as plsc

_NC, _NS = 2, 16
_NW = _NC * _NS
_L = 16


def _sc_mesh():
    return plsc.VectorSubcoreMesh(core_axis_name="c", subcore_axis_name="s")


def _sc_params():
    cp = pltpu.CompilerParams()
    fields = pltpu.CompilerParams.__dataclass_fields__
    if "needs_layout_passes" in fields:
        cp = dataclasses.replace(cp, needs_layout_passes=False)
    if "use_tc_tiling_on_sc" in fields:
        cp = dataclasses.replace(cp, use_tc_tiling_on_sc=False)
    return cp


def _wid():
    return lax.axis_index("s") * _NC + lax.axis_index("c")


def _deg_body(src_hbm, ew_hbm, degp_hbm, sbuf, wbuf, degbuf, *, EW, NP):
    wid = _wid()
    base = wid * EW
    zeros = jnp.zeros((_L,), jnp.float32)

    @pl.loop(0, NP, step=_L)
    def _(i):
        degbuf[pl.ds(i, _L)] = zeros

    pltpu.sync_copy(src_hbm.at[pl.ds(base, EW)], sbuf)
    pltpu.sync_copy(ew_hbm.at[pl.ds(base, EW)], wbuf)

    @plsc.parallel_loop(0, EW, step=_L, unroll=2)
    def _(j):
        s16 = sbuf[pl.ds(j, _L)]
        w16 = wbuf[pl.ds(j, _L)]
        plsc.addupdate_scatter(degbuf, [s16], w16)

    pltpu.sync_copy(degbuf, degp_hbm.at[wid])


def _run_deg(src_p, ew_p, NP):
    E = src_p.shape[0]
    EW = E // _NW
    k = pl.kernel(
        functools.partial(_deg_body, EW=EW, NP=NP),
        out_type=jax.ShapeDtypeStruct((_NW, NP), jnp.float32),
        mesh=_sc_mesh(),
        compiler_params=_sc_params(),
        scratch_types=[
            pltpu.VMEM((EW,), jnp.int32),
            pltpu.VMEM((EW,), jnp.float32),
            pltpu.VMEM((NP,), jnp.float32),
        ],
    )
    return k(src_p, ew_p)


def _dis_body(degp_ref, o_ref):
    d = jnp.sum(degp_ref[...], axis=0, keepdims=True)
    o_ref[...] = jnp.where(d > 0, lax.rsqrt(jnp.where(d > 0, d, 1.0)), 0.0)


def _run_dis(degp, NP):
    out = pl.pallas_call(
        _dis_body,
        out_shape=jax.ShapeDtypeStruct((1, NP), jnp.float32),
    )(degp)
    return out.reshape(NP)


def _wnorm_body(dis_hbm, src_hbm, dst_hbm, ew_hbm, wn_hbm,
                disbuf, sbuf, dbuf, wbuf, obuf, *, EW):
    base = _wid() * EW
    pltpu.sync_copy(dis_hbm, disbuf)
    pltpu.sync_copy(src_hbm.at[pl.ds(base, EW)], sbuf)
    pltpu.sync_copy(dst_hbm.at[pl.ds(base, EW)], dbuf)
    pltpu.sync_copy(ew_hbm.at[pl.ds(base, EW)], wbuf)

    @plsc.parallel_loop(0, EW, step=_L, unroll=2)
    def _(j):
        s16 = sbuf[pl.ds(j, _L)]
        d16 = dbuf[pl.ds(j, _L)]
        w16 = wbuf[pl.ds(j, _L)]
        a = plsc.load_gather(disbuf, [s16])
        b = plsc.load_gather(disbuf, [d16])
        obuf[pl.ds(j, _L)] = -(a * w16) * b

    pltpu.sync_copy(obuf, wn_hbm.at[pl.ds(base, EW)])


def _run_wnorm(dis, src_p, dst_p, ew_p, NP):
    E = src_p.shape[0]
    EW = E // _NW
    k = pl.kernel(
        functools.partial(_wnorm_body, EW=EW),
        out_type=jax.ShapeDtypeStruct((E,), jnp.float32),
        mesh=_sc_mesh(),
        compiler_params=_sc_params(),
        scratch_types=[
            pltpu.VMEM((NP,), jnp.float32),
            pltpu.VMEM((EW,), jnp.int32),
            pltpu.VMEM((EW,), jnp.int32),
            pltpu.VMEM((EW,), jnp.float32),
            pltpu.VMEM((EW,), jnp.float32),
        ],
    )
    return k(dis, src_p, dst_p, ew_p)


def _dg(a, b):
    return lax.dot_general(a, b, (((0,), (0,)), ((), ())),
                           preferred_element_type=jnp.float32)


def _tconv1_body(x0_ref, x1_ref, x2_ref, w_ref, b_ref, o_ref, *, H):
    w = w_ref[...].astype(jnp.bfloat16)
    acc = None
    for k, xr in enumerate((x0_ref, x1_ref, x2_ref)):
        xk = xr[0].astype(jnp.bfloat16)
        p = lax.dot_general(w[k], xk, (((0,), (1,)), ((), ())),
                            preferred_element_type=jnp.float32)
        acc = p if acc is None else acc + p
    acc = acc + b_ref[...]
    a, g, c = acc[:H], acc[H:2 * H], acc[2 * H:]
    o_ref[...] = jax.nn.relu(a + jax.nn.sigmoid(g) + c)


def _run_tconv1(Xp, Wc, bc, T1, H, NP, bw=512):
    KS, C, _ = Wc.shape
    grid = (T1, NP // bw)
    in_specs = [
        pl.BlockSpec((1, bw, C), (lambda t, n, k=k: (t + k, n, 0)))
        for k in range(KS)
    ] + [
        pl.BlockSpec((KS, C, 3 * H), lambda t, n: (0, 0, 0)),
        pl.BlockSpec((3 * H, 1), lambda t, n: (0, 0)),
    ]
    out = pl.pallas_call(
        functools.partial(_tconv1_body, H=H),
        grid=grid,
        in_specs=in_specs,
        out_specs=pl.BlockSpec((H, bw), lambda t, n: (t, n)),
        out_shape=jax.ShapeDtypeStruct((T1 * H, NP), jnp.float32),
    )(Xp, Xp, Xp, Wc, bc)
    return out


def _cheb_mp_body(x_hbm, ep_hbm, y1_hbm, y2_hbm, xbuf, ybuf, eb0, eb1,
                  sem0, sem1, *, NP, EB, NBLK, R, G):
    wid = _wid()

    def mp_pass(src_ref, acc_ref):
        zeros = jnp.zeros((_L,), jnp.float32)
        for r in range(R):
            @pl.loop(0, NP, step=_L)
            def _(i):
                acc_ref[r, pl.ds(i, _L)] = zeros

        def process(ebuf):
            @plsc.parallel_loop(0, EB, step=_L, unroll=2)
            def _(j):
                s16 = ebuf[0, pl.ds(j, _L)]
                d16 = ebuf[1, pl.ds(j, _L)]
                w16 = plsc.bitcast(ebuf[2, pl.ds(j, _L)], jnp.float32)
                for r in range(R):
                    r16 = jnp.full((_L,), r, jnp.int32)
                    zv = plsc.load_gather(src_ref, [r16, s16])
                    plsc.addupdate_scatter(acc_ref, [r16, d16], w16 * zv)

        pltpu.async_copy(ep_hbm.at[0], eb0, sem0)

        @pl.loop(0, NBLK, step=2)
        def _(blk):
            pltpu.async_copy(ep_hbm.at[blk + 1], eb1, sem1)
            pltpu.make_async_copy(ep_hbm.at[blk], eb0, sem0).wait()
            process(eb0)

            @pl.when(blk + 2 < NBLK)
            def _():
                pltpu.async_copy(ep_hbm.at[blk + 2], eb0, sem0)

            pltpu.make_async_copy(ep_hbm.at[blk + 1], eb1, sem1).wait()
            process(eb1)

    for g in range(G):
        row0 = wid * (R * G) + g * R
        pltpu.sync_copy(x_hbm.at[pl.ds(row0, R)], xbuf)
        mp_pass(xbuf, ybuf)
        pltpu.sync_copy(ybuf, y1_hbm.at[pl.ds(row0, R)])
        mp_pass(ybuf, xbuf)
        pltpu.sync_copy(xbuf, y2_hbm.at[pl.ds(row0, R)])


def _run_cheb_mp(Tx0, ep, NP, R, G, EB):
    NBLK = ep.shape[0]
    RT = Tx0.shape[0]
    sds = jax.ShapeDtypeStruct((RT, NP), jnp.float32)
    k = pl.kernel(
        functools.partial(_cheb_mp_body, NP=NP, EB=EB, NBLK=NBLK, R=R, G=G),
        out_type=(sds, sds),
        mesh=_sc_mesh(),
        compiler_params=_sc_params(),
        scratch_types=[
            pltpu.VMEM((R, NP), jnp.float32),
            pltpu.VMEM((R, NP), jnp.float32),
            pltpu.VMEM((3, EB), jnp.int32),
            pltpu.VMEM((3, EB), jnp.int32),
            pltpu.SemaphoreType.DMA,
            pltpu.SemaphoreType.DMA,
        ],
    )
    return k(Tx0, ep)


def _tail_body(tx0_ref, y1_ref, y2_ref, chw_ref, chb_ref, w2_ref, b2_ref,
               g_ref, bt_ref, o_ref, *, T1, T2, H, F):
    A0 = (chw_ref[0] - chw_ref[2]).astype(jnp.bfloat16)
    A1 = chw_ref[1].astype(jnp.bfloat16)
    A2 = (2.0 * chw_ref[2]).astype(jnp.bfloat16)
    chb = chb_ref[...]

    t2 = []
    for t in range(T1):
        z0 = tx0_ref[t * H:(t + 1) * H, :].astype(jnp.bfloat16)
        z1 = y1_ref[t * H:(t + 1) * H, :].astype(jnp.bfloat16)
        z2 = y2_ref[t * H:(t + 1) * H, :].astype(jnp.bfloat16)
        t2.append(jax.nn.relu(_dg(A0, z0) + _dg(A1, z1) + _dg(A2, z2) + chb))

    w2 = w2_ref[...].astype(jnp.bfloat16)
    b2 = b2_ref[...]
    t3 = []
    for tt in range(T2):
        acc = None
        for k in range(w2.shape[0]):
            p = _dg(w2[k], t2[tt + k].astype(jnp.bfloat16))
            acc = p if acc is None else acc + p
        acc = acc + b2
        a, g, c = acc[:F], acc[F:2 * F], acc[2 * F:]
        t3.append(jax.nn.relu(a + jax.nn.sigmoid(g) + c))

    cnt = float(T2 * F)
    m = sum(jnp.sum(x, axis=0) for x in t3) / cnt
    v = sum(jnp.sum((x - m[None, :]) ** 2, axis=0) for x in t3) / cnt
    scale = lax.rsqrt(v + 1e-5) * g_ref[0]
    bet = bt_ref[0]
    for tt in range(T2):
        o_ref[tt * F:(tt + 1) * F, :] = (t3[tt] - m[None, :]) * scale[None, :] + bet[None, :]


def _run_tail(Tx0, Y1, Y2, cheb_w, chb, Wc2, bc2, bn_g, bn_b,
              T1, T2, H, F, NP, bw=512):
    RT = Tx0.shape[0]
    KS = Wc2.shape[0]
    fm = pl.BlockSpec((RT, bw), lambda n: (0, n))
    full = lambda shape: pl.BlockSpec(shape, lambda n: tuple(0 for _ in shape))
    out = pl.pallas_call(
        functools.partial(_tail_body, T1=T1, T2=T2, H=H, F=F),
        grid=(NP // bw,),
        in_specs=[
            fm, fm, fm,
            full((cheb_w.shape[0], H, H)),
            full((H, 1)),
            full((KS, H, 3 * F)),
            full((3 * F, 1)),
            pl.BlockSpec((1, bw), lambda n: (0, n)),
            pl.BlockSpec((1, bw), lambda n: (0, n)),
        ],
        out_specs=pl.BlockSpec((T2 * F, bw), lambda n: (0, n)),
        out_shape=jax.ShapeDtypeStruct((T2 * F, NP), jnp.float32),
    )(Tx0, Y1, Y2, cheb_w, chb, Wc2, bc2, bn_g, bn_b)
    return out


def _round_up(x, m):
    return (x + m - 1) // m * m


def _conv_wstack(w1, w2, w3):
    return jnp.concatenate(
        [jnp.transpose(w[:, :, 0, :], (2, 1, 0)) for w in (w1, w2, w3)],
        axis=2)


def kernel(X, edge_index, edge_weight, tc1_w1, tc1_b1, tc1_w2, tc1_b2,
           tc1_w3, tc1_b3, cheb_w, cheb_b, tc2_w1, tc2_b1, tc2_w2, tc2_b2,
           tc2_w3, tc2_b3, bn_gamma, bn_beta):
    B, T, N, C = X.shape
    H = tc1_w1.shape[0]
    F = tc2_w1.shape[0]
    T1, T2 = T - 2, T - 4
    E = edge_weight.shape[0]

    NP = _round_up(N, 512)
    EB_MP = 2048
    EP = _round_up(E, max(2 * EB_MP, _NW * _L * 8))

    src_p = jnp.pad(edge_index[0], (0, EP - E))
    dst_p = jnp.pad(edge_index[1], (0, EP - E))
    ew_p = jnp.pad(edge_weight, (0, EP - E))

    degp = _run_deg(src_p, ew_p, NP)
    dis = _run_dis(degp, NP)
    wn = _run_wnorm(dis, src_p, dst_p, ew_p, NP)

    wn_i = lax.bitcast_convert_type(wn, jnp.int32)
    ep = jnp.stack([src_p, dst_p, wn_i]).reshape(3, EP // EB_MP, EB_MP)
    ep = jnp.transpose(ep, (1, 0, 2))

    Xp = jnp.pad(X[0], ((0, 0), (0, NP - N), (0, 0)))
    Wc1 = _conv_wstack(tc1_w1, tc1_w2, tc1_w3)
    bc1 = jnp.concatenate([tc1_b1, tc1_b2, tc1_b3])[:, None]
    Tx0 = _run_tconv1(Xp, Wc1, bc1, T1, H, NP)

    RT = T1 * H
    rows_per_w = RT // _NW
    R = 5 if rows_per_w % 5 == 0 else rows_per_w
    G = rows_per_w // R
    Y1, Y2 = _run_cheb_mp(Tx0, ep, NP, R, G, EB_MP)

    Wc2 = _conv_wstack(tc2_w1, tc2_w2, tc2_w3)
    bc2 = jnp.concatenate([tc2_b1, tc2_b2, tc2_b3])[:, None]
    bn_g = jnp.pad(bn_gamma, (0, NP - N))[None, :]
    bn_b = jnp.pad(bn_beta, (0, NP - N))[None, :]
    out_fm = _run_tail(Tx0, Y1, Y2, cheb_w, cheb_b[:, None], Wc2, bc2,
                       bn_g, bn_b, T1, T2, H, F, NP)

    out = out_fm[:, :N].reshape(T2, F, N)
    return jnp.transpose(out, (0, 2, 1))[None]


# speedup vs baseline: 1.2896x; 1.0013x over previous
import dataclasses
import functools

import jax
import jax.numpy as jnp
from jax import lax
from jax.experimental import pallas as pl
from jax.experimental.pallas import tpu as pltpu
from jax.experimental.pallas import tpu_sc as plsc

_NC, _NS = 2, 16
_NW = _NC * _NS
_L = 16


def _sc_mesh():
    return plsc.VectorSubcoreMesh(core_axis_name="c", subcore_axis_name="s")


def _sc_params():
    cp = pltpu.CompilerParams()
    fields = pltpu.CompilerParams.__dataclass_fields__
    if "needs_layout_passes" in fields:
        cp = dataclasses.replace(cp, needs_layout_passes=False)
    if "use_tc_tiling_on_sc" in fields:
        cp = dataclasses.replace(cp, use_tc_tiling_on_sc=False)
    return cp


def _wid():
    return lax.axis_index("s") * _NC + lax.axis_index("c")


def _deg_body(src_hbm, ew_hbm, degp_hbm, sbuf, wbuf, degbuf, *, EW, NP):
    wid = _wid()
    base = wid * EW
    zeros = jnp.zeros((_L,), jnp.float32)

    @pl.loop(0, NP, step=_L)
    def _(i):
        degbuf[pl.ds(i, _L)] = zeros

    pltpu.sync_copy(src_hbm.at[pl.ds(base, EW)], sbuf)
    pltpu.sync_copy(ew_hbm.at[pl.ds(base, EW)], wbuf)

    @plsc.parallel_loop(0, EW, step=_L, unroll=2)
    def _(j):
        s16 = sbuf[pl.ds(j, _L)]
        w16 = wbuf[pl.ds(j, _L)]
        plsc.addupdate_scatter(degbuf, [s16], w16)

    pltpu.sync_copy(degbuf, degp_hbm.at[wid])


def _run_deg(src_p, ew_p, NP):
    E = src_p.shape[0]
    EW = E // _NW
    k = pl.kernel(
        functools.partial(_deg_body, EW=EW, NP=NP),
        out_type=jax.ShapeDtypeStruct((_NW, NP), jnp.float32),
        mesh=_sc_mesh(),
        compiler_params=_sc_params(),
        scratch_types=[
            pltpu.VMEM((EW,), jnp.int32),
            pltpu.VMEM((EW,), jnp.float32),
            pltpu.VMEM((NP,), jnp.float32),
        ],
    )
    return k(src_p, ew_p)


def _dis_body(degp_ref, o_ref):
    d = jnp.sum(degp_ref[...], axis=0, keepdims=True)
    o_ref[...] = jnp.where(d > 0, lax.rsqrt(jnp.where(d > 0, d, 1.0)), 0.0)


def _run_dis(degp, NP):
    out = pl.pallas_call(
        _dis_body,
        out_shape=jax.ShapeDtypeStruct((1, NP), jnp.float32),
    )(degp)
    return out.reshape(NP)


def _wnorm_body(dis_hbm, src_hbm, dst_hbm, ew_hbm, wn_hbm,
                disbuf, sbuf, dbuf, wbuf, obuf, *, EW):
    base = _wid() * EW
    pltpu.sync_copy(dis_hbm, disbuf)
    pltpu.sync_copy(src_hbm.at[pl.ds(base, EW)], sbuf)
    pltpu.sync_copy(dst_hbm.at[pl.ds(base, EW)], dbuf)
    pltpu.sync_copy(ew_hbm.at[pl.ds(base, EW)], wbuf)

    @plsc.parallel_loop(0, EW, step=_L, unroll=2)
    def _(j):
        s16 = sbuf[pl.ds(j, _L)]
        d16 = dbuf[pl.ds(j, _L)]
        w16 = wbuf[pl.ds(j, _L)]
        a = plsc.load_gather(disbuf, [s16])
        b = plsc.load_gather(disbuf, [d16])
        obuf[pl.ds(j, _L)] = -(a * w16) * b

    pltpu.sync_copy(obuf, wn_hbm.at[pl.ds(base, EW)])


def _run_wnorm(dis, src_p, dst_p, ew_p, NP):
    E = src_p.shape[0]
    EW = E // _NW
    k = pl.kernel(
        functools.partial(_wnorm_body, EW=EW),
        out_type=jax.ShapeDtypeStruct((E,), jnp.float32),
        mesh=_sc_mesh(),
        compiler_params=_sc_params(),
        scratch_types=[
            pltpu.VMEM((NP,), jnp.float32),
            pltpu.VMEM((EW,), jnp.int32),
            pltpu.VMEM((EW,), jnp.int32),
            pltpu.VMEM((EW,), jnp.float32),
            pltpu.VMEM((EW,), jnp.float32),
        ],
    )
    return k(dis, src_p, dst_p, ew_p)


def _dg(a, b):
    return lax.dot_general(a, b, (((0,), (0,)), ((), ())),
                           preferred_element_type=jnp.float32)


def _tconv1_body(x0_ref, x1_ref, x2_ref, w_ref, b_ref, o_ref, *, H):
    w = w_ref[...].astype(jnp.bfloat16)
    acc = None
    for k, xr in enumerate((x0_ref, x1_ref, x2_ref)):
        xk = xr[0].astype(jnp.bfloat16)
        p = lax.dot_general(w[k], xk, (((0,), (1,)), ((), ())),
                            preferred_element_type=jnp.float32)
        acc = p if acc is None else acc + p
    acc = acc + b_ref[...]
    a, g, c = acc[:H], acc[H:2 * H], acc[2 * H:]
    o_ref[...] = jax.nn.relu(a + jax.nn.sigmoid(g) + c)


def _run_tconv1(Xp, Wc, bc, T1, H, NP, bw=512):
    KS, C, _ = Wc.shape
    grid = (T1, NP // bw)
    in_specs = [
        pl.BlockSpec((1, bw, C), (lambda t, n, k=k: (t + k, n, 0)))
        for k in range(KS)
    ] + [
        pl.BlockSpec((KS, C, 3 * H), lambda t, n: (0, 0, 0)),
        pl.BlockSpec((3 * H, 1), lambda t, n: (0, 0)),
    ]
    out = pl.pallas_call(
        functools.partial(_tconv1_body, H=H),
        grid=grid,
        in_specs=in_specs,
        out_specs=pl.BlockSpec((H, bw), lambda t, n: (t, n)),
        out_shape=jax.ShapeDtypeStruct((T1 * H, NP), jnp.float32),
    )(Xp, Xp, Xp, Wc, bc)
    return out


def _cheb_mp_body(x_hbm, ep_hbm, y1_hbm, y2_hbm, xbuf, ybuf, eb0, eb1,
                  sem0, sem1, *, NP, EB, NBLK, R, G):
    wid = _wid()

    def mp_pass(src_ref, acc_ref):
        zeros = jnp.zeros((_L,), jnp.float32)
        for r in range(R):
            @pl.loop(0, NP, step=_L)
            def _(i):
                acc_ref[r, pl.ds(i, _L)] = zeros

        def process(ebuf):
            @plsc.parallel_loop(0, EB, step=_L, unroll=2)
            def _(j):
                s16 = ebuf[0, pl.ds(j, _L)]
                d16 = ebuf[1, pl.ds(j, _L)]
                w16 = plsc.bitcast(ebuf[2, pl.ds(j, _L)], jnp.float32)
                for r in range(R):
                    r16 = jnp.full((_L,), r, jnp.int32)
                    zv = plsc.load_gather(src_ref, [r16, s16])
                    plsc.addupdate_scatter(acc_ref, [r16, d16], w16 * zv)

        pltpu.async_copy(ep_hbm.at[0], eb0, sem0)

        @pl.loop(0, NBLK, step=2)
        def _(blk):
            pltpu.async_copy(ep_hbm.at[blk + 1], eb1, sem1)
            pltpu.make_async_copy(ep_hbm.at[blk], eb0, sem0).wait()
            process(eb0)

            @pl.when(blk + 2 < NBLK)
            def _():
                pltpu.async_copy(ep_hbm.at[blk + 2], eb0, sem0)

            pltpu.make_async_copy(ep_hbm.at[blk + 1], eb1, sem1).wait()
            process(eb1)

    for g in range(G):
        row0 = wid * (R * G) + g * R
        pltpu.sync_copy(x_hbm.at[pl.ds(row0, R)], xbuf)
        mp_pass(xbuf, ybuf)
        pltpu.sync_copy(ybuf, y1_hbm.at[pl.ds(row0, R)])
        mp_pass(ybuf, xbuf)
        pltpu.sync_copy(xbuf, y2_hbm.at[pl.ds(row0, R)])


def _run_cheb_mp(Tx0, ep, NP, R, G, EB):
    NBLK = ep.shape[0]
    RT = Tx0.shape[0]
    sds = jax.ShapeDtypeStruct((RT, NP), jnp.float32)
    k = pl.kernel(
        functools.partial(_cheb_mp_body, NP=NP, EB=EB, NBLK=NBLK, R=R, G=G),
        out_type=(sds, sds),
        mesh=_sc_mesh(),
        compiler_params=_sc_params(),
        scratch_types=[
            pltpu.VMEM((R, NP), jnp.float32),
            pltpu.VMEM((R, NP), jnp.float32),
            pltpu.VMEM((3, EB), jnp.int32),
            pltpu.VMEM((3, EB), jnp.int32),
            pltpu.SemaphoreType.DMA,
            pltpu.SemaphoreType.DMA,
        ],
    )
    return k(Tx0, ep)


def _tail_body(tx0_ref, y1_ref, y2_ref, chw_ref, chb_ref, w2_ref, b2_ref,
               g_ref, bt_ref, o_ref, *, T1, T2, H, F):
    A0 = (chw_ref[0] - chw_ref[2]).astype(jnp.bfloat16)
    A1 = chw_ref[1].astype(jnp.bfloat16)
    A2 = (2.0 * chw_ref[2]).astype(jnp.bfloat16)
    chb = chb_ref[...]

    t2 = []
    for t in range(T1):
        z0 = tx0_ref[t * H:(t + 1) * H, :].astype(jnp.bfloat16)
        z1 = y1_ref[t * H:(t + 1) * H, :].astype(jnp.bfloat16)
        z2 = y2_ref[t * H:(t + 1) * H, :].astype(jnp.bfloat16)
        t2.append(jax.nn.relu(_dg(A0, z0) + _dg(A1, z1) + _dg(A2, z2) + chb))

    w2 = w2_ref[...].astype(jnp.bfloat16)
    b2 = b2_ref[...]
    t3 = []
    for tt in range(T2):
        acc = None
        for k in range(w2.shape[0]):
            p = _dg(w2[k], t2[tt + k].astype(jnp.bfloat16))
            acc = p if acc is None else acc + p
        acc = acc + b2
        a, g, c = acc[:F], acc[F:2 * F], acc[2 * F:]
        t3.append(jax.nn.relu(a + jax.nn.sigmoid(g) + c))

    cnt = float(T2 * F)
    m = sum(jnp.sum(x, axis=0) for x in t3) / cnt
    v = sum(jnp.sum((x - m[None, :]) ** 2, axis=0) for x in t3) / cnt
    scale = lax.rsqrt(v + 1e-5) * g_ref[0]
    bet = bt_ref[0]
    for tt in range(T2):
        o_ref[tt * F:(tt + 1) * F, :] = (t3[tt] - m[None, :]) * scale[None, :] + bet[None, :]


def _run_tail(Tx0, Y1, Y2, cheb_w, chb, Wc2, bc2, bn_g, bn_b,
              T1, T2, H, F, NP, bw=512):
    RT = Tx0.shape[0]
    KS = Wc2.shape[0]
    fm = pl.BlockSpec((RT, bw), lambda n: (0, n))
    full = lambda shape: pl.BlockSpec(shape, lambda n: tuple(0 for _ in shape))
    out = pl.pallas_call(
        functools.partial(_tail_body, T1=T1, T2=T2, H=H, F=F),
        grid=(NP // bw,),
        in_specs=[
            fm, fm, fm,
            full((cheb_w.shape[0], H, H)),
            full((H, 1)),
            full((KS, H, 3 * F)),
            full((3 * F, 1)),
            pl.BlockSpec((1, bw), lambda n: (0, n)),
            pl.BlockSpec((1, bw), lambda n: (0, n)),
        ],
        out_specs=pl.BlockSpec((T2 * F, bw), lambda n: (0, n)),
        out_shape=jax.ShapeDtypeStruct((T2 * F, NP), jnp.float32),
    )(Tx0, Y1, Y2, cheb_w, chb, Wc2, bc2, bn_g, bn_b)
    return out


def _round_up(x, m):
    return (x + m - 1) // m * m


def _conv_wstack(w1, w2, w3):
    return jnp.concatenate(
        [jnp.transpose(w[:, :, 0, :], (2, 1, 0)) for w in (w1, w2, w3)],
        axis=2)


def kernel(X, edge_index, edge_weight, tc1_w1, tc1_b1, tc1_w2, tc1_b2,
           tc1_w3, tc1_b3, cheb_w, cheb_b, tc2_w1, tc2_b1, tc2_w2, tc2_b2,
           tc2_w3, tc2_b3, bn_gamma, bn_beta):
    B, T, N, C = X.shape
    H = tc1_w1.shape[0]
    F = tc2_w1.shape[0]
    T1, T2 = T - 2, T - 4
    E = edge_weight.shape[0]

    NP = _round_up(N, 512)
    EB_MP = 4096
    EP = _round_up(E, max(2 * EB_MP, _NW * _L * 8))

    src_p = jnp.pad(edge_index[0], (0, EP - E))
    dst_p = jnp.pad(edge_index[1], (0, EP - E))
    ew_p = jnp.pad(edge_weight, (0, EP - E))

    degp = _run_deg(src_p, ew_p, NP)
    dis = _run_dis(degp, NP)
    wn = _run_wnorm(dis, src_p, dst_p, ew_p, NP)

    wn_i = lax.bitcast_convert_type(wn, jnp.int32)
    ep = jnp.stack([src_p, dst_p, wn_i]).reshape(3, EP // EB_MP, EB_MP)
    ep = jnp.transpose(ep, (1, 0, 2))

    Xp = jnp.pad(X[0], ((0, 0), (0, NP - N), (0, 0)))
    Wc1 = _conv_wstack(tc1_w1, tc1_w2, tc1_w3)
    bc1 = jnp.concatenate([tc1_b1, tc1_b2, tc1_b3])[:, None]
    Tx0 = _run_tconv1(Xp, Wc1, bc1, T1, H, NP)

    RT = T1 * H
    rows_per_w = RT // _NW
    R = 5 if rows_per_w % 5 == 0 else rows_per_w
    G = rows_per_w // R
    Y1, Y2 = _run_cheb_mp(Tx0, ep, NP, R, G, EB_MP)

    Wc2 = _conv_wstack(tc2_w1, tc2_w2, tc2_w3)
    bc2 = jnp.concatenate([tc2_b1, tc2_b2, tc2_b3])[:, None]
    bn_g = jnp.pad(bn_gamma, (0, NP - N))[None, :]
    bn_b = jnp.pad(bn_beta, (0, NP - N))[None, :]
    out_fm = _run_tail(Tx0, Y1, Y2, cheb_w, cheb_b[:, None], Wc2, bc2,
                       bn_g, bn_b, T1, T2, H, F, NP)

    out = out_fm[:, :N].reshape(T2, F, N)
    return jnp.transpose(out, (0, 2, 1))[None]


# speedup vs baseline: 1.2897x; 1.0000x over previous
import dataclasses
import functools

import jax
import jax.numpy as jnp
from jax import lax
from jax.experimental import pallas as pl
from jax.experimental.pallas import tpu as pltpu
from jax.experimental.pallas import tpu_sc as plsc

_NC, _NS = 2, 16
_NW = _NC * _NS
_L = 16


def _sc_mesh():
    return plsc.VectorSubcoreMesh(core_axis_name="c", subcore_axis_name="s")


def _sc_params():
    cp = pltpu.CompilerParams()
    fields = pltpu.CompilerParams.__dataclass_fields__
    if "needs_layout_passes" in fields:
        cp = dataclasses.replace(cp, needs_layout_passes=False)
    if "use_tc_tiling_on_sc" in fields:
        cp = dataclasses.replace(cp, use_tc_tiling_on_sc=False)
    return cp


def _wid():
    return lax.axis_index("s") * _NC + lax.axis_index("c")


def _deg_body(src_hbm, ew_hbm, degp_hbm, sbuf, wbuf, degbuf, *, EW, NP):
    wid = _wid()
    base = wid * EW
    zeros = jnp.zeros((_L,), jnp.float32)

    @pl.loop(0, NP, step=_L)
    def _(i):
        degbuf[pl.ds(i, _L)] = zeros

    pltpu.sync_copy(src_hbm.at[pl.ds(base, EW)], sbuf)
    pltpu.sync_copy(ew_hbm.at[pl.ds(base, EW)], wbuf)

    @plsc.parallel_loop(0, EW, step=_L, unroll=2)
    def _(j):
        s16 = sbuf[pl.ds(j, _L)]
        w16 = wbuf[pl.ds(j, _L)]
        plsc.addupdate_scatter(degbuf, [s16], w16)

    pltpu.sync_copy(degbuf, degp_hbm.at[wid])


def _run_deg(src_p, ew_p, NP):
    E = src_p.shape[0]
    EW = E // _NW
    k = pl.kernel(
        functools.partial(_deg_body, EW=EW, NP=NP),
        out_type=jax.ShapeDtypeStruct((_NW, NP), jnp.float32),
        mesh=_sc_mesh(),
        compiler_params=_sc_params(),
        scratch_types=[
            pltpu.VMEM((EW,), jnp.int32),
            pltpu.VMEM((EW,), jnp.float32),
            pltpu.VMEM((NP,), jnp.float32),
        ],
    )
    return k(src_p, ew_p)


def _dis_body(degp_ref, o_ref):
    d = jnp.sum(degp_ref[...], axis=0, keepdims=True)
    o_ref[...] = jnp.where(d > 0, lax.rsqrt(jnp.where(d > 0, d, 1.0)), 0.0)


def _run_dis(degp, NP):
    out = pl.pallas_call(
        _dis_body,
        out_shape=jax.ShapeDtypeStruct((1, NP), jnp.float32),
    )(degp)
    return out.reshape(NP)


def _wnorm_body(dis_hbm, src_hbm, dst_hbm, ew_hbm, wn_hbm,
                disbuf, sbuf, dbuf, wbuf, obuf, *, EW):
    base = _wid() * EW
    pltpu.sync_copy(dis_hbm, disbuf)
    pltpu.sync_copy(src_hbm.at[pl.ds(base, EW)], sbuf)
    pltpu.sync_copy(dst_hbm.at[pl.ds(base, EW)], dbuf)
    pltpu.sync_copy(ew_hbm.at[pl.ds(base, EW)], wbuf)

    @plsc.parallel_loop(0, EW, step=_L, unroll=2)
    def _(j):
        s16 = sbuf[pl.ds(j, _L)]
        d16 = dbuf[pl.ds(j, _L)]
        w16 = wbuf[pl.ds(j, _L)]
        a = plsc.load_gather(disbuf, [s16])
        b = plsc.load_gather(disbuf, [d16])
        obuf[pl.ds(j, _L)] = -(a * w16) * b

    pltpu.sync_copy(obuf, wn_hbm.at[pl.ds(base, EW)])


def _run_wnorm(dis, src_p, dst_p, ew_p, NP):
    E = src_p.shape[0]
    EW = E // _NW
    k = pl.kernel(
        functools.partial(_wnorm_body, EW=EW),
        out_type=jax.ShapeDtypeStruct((E,), jnp.float32),
        mesh=_sc_mesh(),
        compiler_params=_sc_params(),
        scratch_types=[
            pltpu.VMEM((NP,), jnp.float32),
            pltpu.VMEM((EW,), jnp.int32),
            pltpu.VMEM((EW,), jnp.int32),
            pltpu.VMEM((EW,), jnp.float32),
            pltpu.VMEM((EW,), jnp.float32),
        ],
    )
    return k(dis, src_p, dst_p, ew_p)


def _dg(a, b):
    return lax.dot_general(a, b, (((0,), (0,)), ((), ())),
                           preferred_element_type=jnp.float32)


def _tconv1_body(x0_ref, x1_ref, x2_ref, w_ref, b_ref, o_ref, *, H):
    w = w_ref[...].astype(jnp.bfloat16)
    acc = None
    for k, xr in enumerate((x0_ref, x1_ref, x2_ref)):
        xk = xr[0].astype(jnp.bfloat16)
        p = lax.dot_general(w[k], xk, (((0,), (1,)), ((), ())),
                            preferred_element_type=jnp.float32)
        acc = p if acc is None else acc + p
    acc = acc + b_ref[...]
    a, g, c = acc[:H], acc[H:2 * H], acc[2 * H:]
    o_ref[...] = jax.nn.relu(a + jax.nn.sigmoid(g) + c)


def _run_tconv1(Xp, Wc, bc, T1, H, NP, bw=512):
    KS, C, _ = Wc.shape
    grid = (T1, NP // bw)
    in_specs = [
        pl.BlockSpec((1, bw, C), (lambda t, n, k=k: (t + k, n, 0)))
        for k in range(KS)
    ] + [
        pl.BlockSpec((KS, C, 3 * H), lambda t, n: (0, 0, 0)),
        pl.BlockSpec((3 * H, 1), lambda t, n: (0, 0)),
    ]
    out = pl.pallas_call(
        functools.partial(_tconv1_body, H=H),
        grid=grid,
        in_specs=in_specs,
        out_specs=pl.BlockSpec((H, bw), lambda t, n: (t, n)),
        out_shape=jax.ShapeDtypeStruct((T1 * H, NP), jnp.float32),
        compiler_params=pltpu.CompilerParams(
            dimension_semantics=("parallel", "parallel")),
    )(Xp, Xp, Xp, Wc, bc)
    return out


def _cheb_mp_body(x_hbm, ep_hbm, y1_hbm, y2_hbm, xbuf, ybuf, eb0, eb1,
                  sem0, sem1, *, NP, EB, NBLK, R, G):
    wid = _wid()

    def mp_pass(src_ref, acc_ref):
        zeros = jnp.zeros((_L,), jnp.float32)
        for r in range(R):
            @pl.loop(0, NP, step=_L)
            def _(i):
                acc_ref[r, pl.ds(i, _L)] = zeros

        def process(ebuf):
            @plsc.parallel_loop(0, EB, step=_L, unroll=2)
            def _(j):
                s16 = ebuf[0, pl.ds(j, _L)]
                d16 = ebuf[1, pl.ds(j, _L)]
                w16 = plsc.bitcast(ebuf[2, pl.ds(j, _L)], jnp.float32)
                for r in range(R):
                    r16 = jnp.full((_L,), r, jnp.int32)
                    zv = plsc.load_gather(src_ref, [r16, s16])
                    plsc.addupdate_scatter(acc_ref, [r16, d16], w16 * zv)

        pltpu.async_copy(ep_hbm.at[0], eb0, sem0)

        @pl.loop(0, NBLK, step=2)
        def _(blk):
            pltpu.async_copy(ep_hbm.at[blk + 1], eb1, sem1)
            pltpu.make_async_copy(ep_hbm.at[blk], eb0, sem0).wait()
            process(eb0)

            @pl.when(blk + 2 < NBLK)
            def _():
                pltpu.async_copy(ep_hbm.at[blk + 2], eb0, sem0)

            pltpu.make_async_copy(ep_hbm.at[blk + 1], eb1, sem1).wait()
            process(eb1)

    for g in range(G):
        row0 = wid * (R * G) + g * R
        pltpu.sync_copy(x_hbm.at[pl.ds(row0, R)], xbuf)
        mp_pass(xbuf, ybuf)
        pltpu.sync_copy(ybuf, y1_hbm.at[pl.ds(row0, R)])
        mp_pass(ybuf, xbuf)
        pltpu.sync_copy(xbuf, y2_hbm.at[pl.ds(row0, R)])


def _run_cheb_mp(Tx0, ep, NP, R, G, EB):
    NBLK = ep.shape[0]
    RT = Tx0.shape[0]
    sds = jax.ShapeDtypeStruct((RT, NP), jnp.float32)
    k = pl.kernel(
        functools.partial(_cheb_mp_body, NP=NP, EB=EB, NBLK=NBLK, R=R, G=G),
        out_type=(sds, sds),
        mesh=_sc_mesh(),
        compiler_params=_sc_params(),
        scratch_types=[
            pltpu.VMEM((R, NP), jnp.float32),
            pltpu.VMEM((R, NP), jnp.float32),
            pltpu.VMEM((3, EB), jnp.int32),
            pltpu.VMEM((3, EB), jnp.int32),
            pltpu.SemaphoreType.DMA,
            pltpu.SemaphoreType.DMA,
        ],
    )
    return k(Tx0, ep)


def _tail_body(tx0_ref, y1_ref, y2_ref, chw_ref, chb_ref, w2_ref, b2_ref,
               g_ref, bt_ref, o_ref, *, T1, T2, H, F):
    A0 = (chw_ref[0] - chw_ref[2]).astype(jnp.bfloat16)
    A1 = chw_ref[1].astype(jnp.bfloat16)
    A2 = (2.0 * chw_ref[2]).astype(jnp.bfloat16)
    chb = chb_ref[...]

    t2 = []
    for t in range(T1):
        z0 = tx0_ref[t * H:(t + 1) * H, :].astype(jnp.bfloat16)
        z1 = y1_ref[t * H:(t + 1) * H, :].astype(jnp.bfloat16)
        z2 = y2_ref[t * H:(t + 1) * H, :].astype(jnp.bfloat16)
        t2.append(jax.nn.relu(_dg(A0, z0) + _dg(A1, z1) + _dg(A2, z2) + chb))

    w2 = w2_ref[...].astype(jnp.bfloat16)
    b2 = b2_ref[...]
    t3 = []
    for tt in range(T2):
        acc = None
        for k in range(w2.shape[0]):
            p = _dg(w2[k], t2[tt + k].astype(jnp.bfloat16))
            acc = p if acc is None else acc + p
        acc = acc + b2
        a, g, c = acc[:F], acc[F:2 * F], acc[2 * F:]
        t3.append(jax.nn.relu(a + jax.nn.sigmoid(g) + c))

    cnt = float(T2 * F)
    m = sum(jnp.sum(x, axis=0) for x in t3) / cnt
    v = sum(jnp.sum((x - m[None, :]) ** 2, axis=0) for x in t3) / cnt
    scale = lax.rsqrt(v + 1e-5) * g_ref[0]
    bet = bt_ref[0]
    for tt in range(T2):
        o_ref[tt * F:(tt + 1) * F, :] = (t3[tt] - m[None, :]) * scale[None, :] + bet[None, :]


def _run_tail(Tx0, Y1, Y2, cheb_w, chb, Wc2, bc2, bn_g, bn_b,
              T1, T2, H, F, NP, bw=512):
    RT = Tx0.shape[0]
    KS = Wc2.shape[0]
    fm = pl.BlockSpec((RT, bw), lambda n: (0, n))
    full = lambda shape: pl.BlockSpec(shape, lambda n: tuple(0 for _ in shape))
    out = pl.pallas_call(
        functools.partial(_tail_body, T1=T1, T2=T2, H=H, F=F),
        grid=(NP // bw,),
        in_specs=[
            fm, fm, fm,
            full((cheb_w.shape[0], H, H)),
            full((H, 1)),
            full((KS, H, 3 * F)),
            full((3 * F, 1)),
            pl.BlockSpec((1, bw), lambda n: (0, n)),
            pl.BlockSpec((1, bw), lambda n: (0, n)),
        ],
        out_specs=pl.BlockSpec((T2 * F, bw), lambda n: (0, n)),
        out_shape=jax.ShapeDtypeStruct((T2 * F, NP), jnp.float32),
        compiler_params=pltpu.CompilerParams(
            dimension_semantics=("parallel",)),
    )(Tx0, Y1, Y2, cheb_w, chb, Wc2, bc2, bn_g, bn_b)
    return out


def _round_up(x, m):
    return (x + m - 1) // m * m


def _conv_wstack(w1, w2, w3):
    return jnp.concatenate(
        [jnp.transpose(w[:, :, 0, :], (2, 1, 0)) for w in (w1, w2, w3)],
        axis=2)


def kernel(X, edge_index, edge_weight, tc1_w1, tc1_b1, tc1_w2, tc1_b2,
           tc1_w3, tc1_b3, cheb_w, cheb_b, tc2_w1, tc2_b1, tc2_w2, tc2_b2,
           tc2_w3, tc2_b3, bn_gamma, bn_beta):
    B, T, N, C = X.shape
    H = tc1_w1.shape[0]
    F = tc2_w1.shape[0]
    T1, T2 = T - 2, T - 4
    E = edge_weight.shape[0]

    NP = _round_up(N, 512)
    EB_MP = 4096
    EP = _round_up(E, max(2 * EB_MP, _NW * _L * 8))

    src_p = jnp.pad(edge_index[0], (0, EP - E))
    dst_p = jnp.pad(edge_index[1], (0, EP - E))
    ew_p = jnp.pad(edge_weight, (0, EP - E))

    degp = _run_deg(src_p, ew_p, NP)
    dis = _run_dis(degp, NP)
    wn = _run_wnorm(dis, src_p, dst_p, ew_p, NP)

    wn_i = lax.bitcast_convert_type(wn, jnp.int32)
    ep = jnp.stack([src_p, dst_p, wn_i]).reshape(3, EP // EB_MP, EB_MP)
    ep = jnp.transpose(ep, (1, 0, 2))

    Xp = jnp.pad(X[0], ((0, 0), (0, NP - N), (0, 0)))
    Wc1 = _conv_wstack(tc1_w1, tc1_w2, tc1_w3)
    bc1 = jnp.concatenate([tc1_b1, tc1_b2, tc1_b3])[:, None]
    Tx0 = _run_tconv1(Xp, Wc1, bc1, T1, H, NP)

    RT = T1 * H
    rows_per_w = RT // _NW
    R = 5 if rows_per_w % 5 == 0 else rows_per_w
    G = rows_per_w // R
    Y1, Y2 = _run_cheb_mp(Tx0, ep, NP, R, G, EB_MP)

    Wc2 = _conv_wstack(tc2_w1, tc2_w2, tc2_w3)
    bc2 = jnp.concatenate([tc2_b1, tc2_b2, tc2_b3])[:, None]
    bn_g = jnp.pad(bn_gamma, (0, NP - N))[None, :]
    bn_b = jnp.pad(bn_beta, (0, NP - N))[None, :]
    out_fm = _run_tail(Tx0, Y1, Y2, cheb_w, cheb_b[:, None], Wc2, bc2,
                       bn_g, bn_b, T1, T2, H, F, NP)

    out = out_fm[:, :N].reshape(T2, F, N)
    return jnp.transpose(out, (0, 2, 1))[None]
